# speedup vs baseline: 2.0796x; 2.0796x over previous
import functools

import jax
import jax.numpy as jnp
from jax import lax
from jax.experimental import pallas as pl
from jax.experimental.pallas import tpu as pltpu
from jax.experimental.pallas import tpu_sc as plsc

_B, _S, _D = 4, 8192, 768
_N = _B * _S
_LN_EPS = 1e-12

_NUM_WORKERS = 32
_ROWS_PER_W = _N // _NUM_WORKERS
_CHUNK = 128
_NCHUNK = _ROWS_PER_W // _CHUNK


def _sc_gather(table, idx):
    mesh = plsc.VectorSubcoreMesh(core_axis_name="c", subcore_axis_name="s")

    @functools.partial(
        pl.kernel,
        out_type=jax.ShapeDtypeStruct((_N, _D), jnp.float32),
        mesh=mesh,
        scratch_types=[
            pltpu.VMEM((_ROWS_PER_W,), jnp.int32),
            pltpu.VMEM((_CHUNK, _D), jnp.float32),
            pltpu.SemaphoreType.DMA,
        ],
    )
    def k(table_hbm, idx_hbm, out_hbm, idx_v, rows_v, sem):
        nc = plsc.get_sparse_core_info().num_cores
        wid = lax.axis_index("s") * nc + lax.axis_index("c")
        base = wid * _ROWS_PER_W
        pltpu.sync_copy(idx_hbm.at[pl.ds(base, _ROWS_PER_W)], idx_v)

        def body(c):
            pltpu.async_copy(
                table_hbm.at[idx_v.at[pl.ds(c * _CHUNK, _CHUNK)]], rows_v, sem
            ).wait()
            pltpu.sync_copy(rows_v, out_hbm.at[pl.ds(base + c * _CHUNK, _CHUNK)])

        pl.loop(0, _NCHUNK)(body)

    return k(table, idx)


_BLK = 256


def _tc_addln_body(inp_ref, pos_ref, tt_ref, trow_ref, gam_ref, bet_ref, out_ref):
    x = inp_ref[...] + pos_ref[...]
    tt = tt_ref[...]
    r0 = trow_ref[0:1, :]
    r1 = trow_ref[1:2, :]
    x = x + r0 + tt * (r1 - r0)
    mean = jnp.mean(x, axis=-1, keepdims=True)
    xc = x - mean
    var = jnp.mean(xc * xc, axis=-1, keepdims=True)
    y = xc * lax.rsqrt(var + _LN_EPS)
    out_ref[...] = y * gam_ref[...] + bet_ref[...]


def _tc_addln(inputs2d, pos2d, ttf, type_table, gamma2d, beta2d):
    grid = (_N // _BLK,)
    return pl.pallas_call(
        _tc_addln_body,
        grid=grid,
        in_specs=[
            pl.BlockSpec((_BLK, _D), lambda i: (i, 0)),
            pl.BlockSpec((_BLK, _D), lambda i: (i, 0)),
            pl.BlockSpec((_BLK, 1), lambda i: (i, 0)),
            pl.BlockSpec((2, _D), lambda i: (0, 0)),
            pl.BlockSpec((1, _D), lambda i: (0, 0)),
            pl.BlockSpec((1, _D), lambda i: (0, 0)),
        ],
        out_specs=pl.BlockSpec((_BLK, _D), lambda i: (i, 0)),
        out_shape=jax.ShapeDtypeStruct((_N, _D), jnp.float32),
    )(inputs2d, pos2d, ttf, type_table, gamma2d, beta2d)


@jax.jit
def kernel(inputs_embeds, position_ids, token_type_ids, pos_table, type_table,
           ln_gamma, ln_beta):
    idx = position_ids.reshape(_N)
    pos2d = _sc_gather(pos_table, idx)
    inputs2d = inputs_embeds.reshape(_N, _D)
    ttf = token_type_ids.reshape(_N, 1).astype(jnp.float32)
    out2d = _tc_addln(inputs2d, pos2d, ttf, type_table,
                      ln_gamma.reshape(1, _D), ln_beta.reshape(1, _D))
    return out2d.reshape(_B, _S, _D)


# speedup vs baseline: 2.4526x; 1.1794x over previous
import functools

import jax
import jax.numpy as jnp
from jax import lax
from jax.experimental import pallas as pl
from jax.experimental.pallas import tpu as pltpu
from jax.experimental.pallas import tpu_sc as plsc

_B, _S, _D = 4, 8192, 768
_N = _B * _S
_LN_EPS = 1e-12

_NUM_WORKERS = 32
_ROWS_PER_W = _N // _NUM_WORKERS
_CHUNK = 128
_NCHUNK = _ROWS_PER_W // _CHUNK


def _sc_gather(table, idx):
    mesh = plsc.VectorSubcoreMesh(core_axis_name="c", subcore_axis_name="s")

    @functools.partial(
        pl.kernel,
        out_type=jax.ShapeDtypeStruct((_N, _D), jnp.float32),
        mesh=mesh,
        scratch_types=[
            pltpu.VMEM((_ROWS_PER_W,), jnp.int32),
            pltpu.VMEM((_CHUNK, _D), jnp.float32),
            pltpu.SemaphoreType.DMA,
        ],
    )
    def k(table_hbm, idx_hbm, out_hbm, idx_v, rows_v, sem):
        nc = plsc.get_sparse_core_info().num_cores
        wid = lax.axis_index("s") * nc + lax.axis_index("c")
        base = wid * _ROWS_PER_W
        pltpu.sync_copy(idx_hbm.at[pl.ds(base, _ROWS_PER_W)], idx_v)

        def body(c):
            pltpu.async_copy(
                table_hbm.at[idx_v.at[pl.ds(c * _CHUNK, _CHUNK)]], rows_v, sem
            ).wait()
            pltpu.sync_copy(rows_v, out_hbm.at[pl.ds(base + c * _CHUNK, _CHUNK)])

        pl.loop(0, _NCHUNK)(body)

    return k(table, idx)


_BLK = 512


def _tc_addln_body(inp_ref, pos_ref, tt_ref, trow_ref, gam_ref, bet_ref, out_ref):
    x = inp_ref[...] + pos_ref[...]
    tt = tt_ref[...]
    r0 = trow_ref[0:1, :]
    r1 = trow_ref[1:2, :]
    x = x + r0 + tt * (r1 - r0)
    mean = jnp.mean(x, axis=-1, keepdims=True)
    xc = x - mean
    var = jnp.mean(xc * xc, axis=-1, keepdims=True)
    y = xc * lax.rsqrt(var + _LN_EPS)
    out_ref[...] = y * gam_ref[...] + bet_ref[...]


def _tc_addln(inputs2d, pos2d, ttf, type_table, gamma2d, beta2d):
    grid = (_N // _BLK,)
    return pl.pallas_call(
        _tc_addln_body,
        grid=grid,
        in_specs=[
            pl.BlockSpec((_BLK, _D), lambda i: (i, 0)),
            pl.BlockSpec((_BLK, _D), lambda i: (i, 0)),
            pl.BlockSpec((_BLK, 1), lambda i: (i, 0)),
            pl.BlockSpec((2, _D), lambda i: (0, 0)),
            pl.BlockSpec((1, _D), lambda i: (0, 0)),
            pl.BlockSpec((1, _D), lambda i: (0, 0)),
        ],
        out_specs=pl.BlockSpec((_BLK, _D), lambda i: (i, 0)),
        out_shape=jax.ShapeDtypeStruct((_N, _D), jnp.float32),
    )(inputs2d, pos2d, ttf, type_table, gamma2d, beta2d)


@jax.jit
def kernel(inputs_embeds, position_ids, token_type_ids, pos_table, type_table,
           ln_gamma, ln_beta):
    idx = position_ids.reshape(_N)
    pos2d = _sc_gather(pos_table, idx)
    inputs2d = inputs_embeds.reshape(_N, _D)
    ttf = token_type_ids.reshape(_N, 1).astype(jnp.float32)
    out2d = _tc_addln(inputs2d, pos2d, ttf, type_table,
                      ln_gamma.reshape(1, _D), ln_beta.reshape(1, _D))
    return out2d.reshape(_B, _S, _D)


# speedup vs baseline: 2.5775x; 1.0509x over previous
import functools

import jax
import jax.numpy as jnp
from jax import lax
from jax.experimental import pallas as pl
from jax.experimental.pallas import tpu as pltpu
from jax.experimental.pallas import tpu_sc as plsc

_B, _S, _D = 4, 8192, 768
_N = _B * _S
_LN_EPS = 1e-12

_NUM_WORKERS = 32
_ROWS_PER_W = _N // _NUM_WORKERS
_CHUNK = 128
_NCHUNK = _ROWS_PER_W // _CHUNK


def _sc_gather(table, idx):
    mesh = plsc.VectorSubcoreMesh(core_axis_name="c", subcore_axis_name="s")

    @functools.partial(
        pl.kernel,
        out_type=jax.ShapeDtypeStruct((_N, _D), jnp.float32),
        mesh=mesh,
        scratch_types=[
            pltpu.VMEM((_ROWS_PER_W,), jnp.int32),
            pltpu.VMEM((_CHUNK, _D), jnp.float32),
            pltpu.SemaphoreType.DMA,
        ],
    )
    def k(table_hbm, idx_hbm, out_hbm, idx_v, rows_v, sem):
        nc = plsc.get_sparse_core_info().num_cores
        wid = lax.axis_index("s") * nc + lax.axis_index("c")
        base = wid * _ROWS_PER_W
        pltpu.sync_copy(idx_hbm.at[pl.ds(base, _ROWS_PER_W)], idx_v)

        def body(c):
            pltpu.async_copy(
                table_hbm.at[idx_v.at[pl.ds(c * _CHUNK, _CHUNK)]], rows_v, sem
            ).wait()
            pltpu.sync_copy(rows_v, out_hbm.at[pl.ds(base + c * _CHUNK, _CHUNK)])

        pl.loop(0, _NCHUNK)(body)

    return k(table, idx)


_BLK = 1024


def _tc_addln_body(inp_ref, pos_ref, tt_ref, trow_ref, gam_ref, bet_ref, out_ref):
    x = inp_ref[...] + pos_ref[...]
    tt = tt_ref[...]
    r0 = trow_ref[0:1, :]
    r1 = trow_ref[1:2, :]
    x = x + r0 + tt * (r1 - r0)
    mean = jnp.mean(x, axis=-1, keepdims=True)
    xc = x - mean
    var = jnp.mean(xc * xc, axis=-1, keepdims=True)
    y = xc * lax.rsqrt(var + _LN_EPS)
    out_ref[...] = y * gam_ref[...] + bet_ref[...]


def _tc_addln(inputs2d, pos2d, ttf, type_table, gamma2d, beta2d):
    grid = (_N // _BLK,)
    return pl.pallas_call(
        _tc_addln_body,
        grid=grid,
        in_specs=[
            pl.BlockSpec((_BLK, _D), lambda i: (i, 0)),
            pl.BlockSpec((_BLK, _D), lambda i: (i, 0)),
            pl.BlockSpec((_BLK, 1), lambda i: (i, 0)),
            pl.BlockSpec((2, _D), lambda i: (0, 0)),
            pl.BlockSpec((1, _D), lambda i: (0, 0)),
            pl.BlockSpec((1, _D), lambda i: (0, 0)),
        ],
        out_specs=pl.BlockSpec((_BLK, _D), lambda i: (i, 0)),
        out_shape=jax.ShapeDtypeStruct((_N, _D), jnp.float32),
    )(inputs2d, pos2d, ttf, type_table, gamma2d, beta2d)


@jax.jit
def kernel(inputs_embeds, position_ids, token_type_ids, pos_table, type_table,
           ln_gamma, ln_beta):
    idx = position_ids.reshape(_N)
    pos2d = _sc_gather(pos_table, idx)
    inputs2d = inputs_embeds.reshape(_N, _D)
    ttf = token_type_ids.reshape(_N, 1).astype(jnp.float32)
    out2d = _tc_addln(inputs2d, pos2d, ttf, type_table,
                      ln_gamma.reshape(1, _D), ln_beta.reshape(1, _D))
    return out2d.reshape(_B, _S, _D)


# speedup vs baseline: 2.6231x; 1.0177x over previous
import functools

import jax
import jax.numpy as jnp
from jax import lax
from jax.experimental import pallas as pl
from jax.experimental.pallas import tpu as pltpu
from jax.experimental.pallas import tpu_sc as plsc

_B, _S, _D = 4, 8192, 768
_N = _B * _S
_LN_EPS = 1e-12

_NUM_WORKERS = 32
_ROWS_PER_W = _N // _NUM_WORKERS
_CHUNK = 128
_NCHUNK = _ROWS_PER_W // _CHUNK


def _sc_gather(table, idx):
    mesh = plsc.VectorSubcoreMesh(core_axis_name="c", subcore_axis_name="s")

    @functools.partial(
        pl.kernel,
        out_type=jax.ShapeDtypeStruct((_N, _D), jnp.float32),
        mesh=mesh,
        scratch_types=[
            pltpu.VMEM((_ROWS_PER_W,), jnp.int32),
            pltpu.VMEM((_CHUNK, _D), jnp.float32),
            pltpu.SemaphoreType.DMA,
        ],
    )
    def k(table_hbm, idx_hbm, out_hbm, idx_v, rows_v, sem):
        nc = plsc.get_sparse_core_info().num_cores
        wid = lax.axis_index("s") * nc + lax.axis_index("c")
        base = wid * _ROWS_PER_W
        pltpu.sync_copy(idx_hbm.at[pl.ds(base, _ROWS_PER_W)], idx_v)

        def body(c):
            pltpu.async_copy(
                table_hbm.at[idx_v.at[pl.ds(c * _CHUNK, _CHUNK)]], rows_v, sem
            ).wait()
            pltpu.sync_copy(rows_v, out_hbm.at[pl.ds(base + c * _CHUNK, _CHUNK)])

        pl.loop(0, _NCHUNK)(body)

    return k(table, idx)


_BLK = 2048


def _tc_addln_body(inp_ref, pos_ref, tt_ref, trow_ref, gam_ref, bet_ref, out_ref):
    x = inp_ref[...] + pos_ref[...]
    tt = tt_ref[...]
    r0 = trow_ref[0:1, :]
    r1 = trow_ref[1:2, :]
    x = x + r0 + tt * (r1 - r0)
    mean = jnp.mean(x, axis=-1, keepdims=True)
    xc = x - mean
    var = jnp.mean(xc * xc, axis=-1, keepdims=True)
    y = xc * lax.rsqrt(var + _LN_EPS)
    out_ref[...] = y * gam_ref[...] + bet_ref[...]


def _tc_addln(inputs2d, pos2d, ttf, type_table, gamma2d, beta2d):
    grid = (_N // _BLK,)
    return pl.pallas_call(
        _tc_addln_body,
        grid=grid,
        in_specs=[
            pl.BlockSpec((_BLK, _D), lambda i: (i, 0)),
            pl.BlockSpec((_BLK, _D), lambda i: (i, 0)),
            pl.BlockSpec((_BLK, 1), lambda i: (i, 0)),
            pl.BlockSpec((2, _D), lambda i: (0, 0)),
            pl.BlockSpec((1, _D), lambda i: (0, 0)),
            pl.BlockSpec((1, _D), lambda i: (0, 0)),
        ],
        out_specs=pl.BlockSpec((_BLK, _D), lambda i: (i, 0)),
        out_shape=jax.ShapeDtypeStruct((_N, _D), jnp.float32),
    )(inputs2d, pos2d, ttf, type_table, gamma2d, beta2d)


@jax.jit
def kernel(inputs_embeds, position_ids, token_type_ids, pos_table, type_table,
           ln_gamma, ln_beta):
    idx = position_ids.reshape(_N)
    pos2d = _sc_gather(pos_table, idx)
    inputs2d = inputs_embeds.reshape(_N, _D)
    ttf = token_type_ids.reshape(_N, 1).astype(jnp.float32)
    out2d = _tc_addln(inputs2d, pos2d, ttf, type_table,
                      ln_gamma.reshape(1, _D), ln_beta.reshape(1, _D))
    return out2d.reshape(_B, _S, _D)


# speedup vs baseline: 2.6532x; 1.0114x over previous
import functools

import jax
import jax.numpy as jnp
from jax import lax
from jax.experimental import pallas as pl
from jax.experimental.pallas import tpu as pltpu
from jax.experimental.pallas import tpu_sc as plsc

_B, _S, _D = 4, 8192, 768
_N = _B * _S
_LN_EPS = 1e-12

_NUM_WORKERS = 32
_ROWS_PER_W = _N // _NUM_WORKERS
_CHUNK = 64
_NCHUNK = _ROWS_PER_W // _CHUNK


def _sc_gather(table, idx):
    mesh = plsc.VectorSubcoreMesh(core_axis_name="c", subcore_axis_name="s")

    @functools.partial(
        pl.kernel,
        out_type=jax.ShapeDtypeStruct((_N, _D), jnp.float32),
        mesh=mesh,
        scratch_types=[
            pltpu.VMEM((_ROWS_PER_W,), jnp.int32),
            pltpu.VMEM((_CHUNK, _D), jnp.float32),
            pltpu.VMEM((_CHUNK, _D), jnp.float32),
            pltpu.SemaphoreType.DMA,
            pltpu.SemaphoreType.DMA,
        ],
    )
    def k(table_hbm, idx_hbm, out_hbm, idx_v, buf0, buf1, sem0, sem1):
        nc = plsc.get_sparse_core_info().num_cores
        wid = lax.axis_index("s") * nc + lax.axis_index("c")
        base = wid * _ROWS_PER_W
        bufs = (buf0, buf1)
        sems = (sem0, sem1)
        pltpu.sync_copy(idx_hbm.at[pl.ds(base, _ROWS_PER_W)], idx_v)

        def gather(c, b):
            pltpu.async_copy(
                table_hbm.at[idx_v.at[pl.ds(c * _CHUNK, _CHUNK)]], bufs[b], sems[b]
            )

        def drain(c, b):
            pltpu.make_async_copy(
                table_hbm.at[idx_v.at[pl.ds(0, _CHUNK)]], bufs[b], sems[b]
            ).wait()
            pltpu.sync_copy(bufs[b], out_hbm.at[pl.ds(base + c * _CHUNK, _CHUNK)])

        gather(0, 0)
        gather(1, 1)

        def body(g):
            for b in range(2):
                c = g + b
                drain(c, b)
                gather(c + 2, b)

        pl.loop(0, _NCHUNK - 2, step=2)(body)
        drain(_NCHUNK - 2, 0)
        drain(_NCHUNK - 1, 1)

    return k(table, idx)


_BLK = 2048


def _tc_addln_body(inp_ref, pos_ref, tt_ref, trow_ref, gam_ref, bet_ref, out_ref):
    x = inp_ref[...] + pos_ref[...]
    tt = tt_ref[...]
    r0 = trow_ref[0:1, :]
    r1 = trow_ref[1:2, :]
    x = x + r0 + tt * (r1 - r0)
    mean = jnp.mean(x, axis=-1, keepdims=True)
    xc = x - mean
    var = jnp.mean(xc * xc, axis=-1, keepdims=True)
    y = xc * lax.rsqrt(var + _LN_EPS)
    out_ref[...] = y * gam_ref[...] + bet_ref[...]


def _tc_addln(inputs2d, pos2d, ttf, type_table, gamma2d, beta2d):
    grid = (_N // _BLK,)
    return pl.pallas_call(
        _tc_addln_body,
        grid=grid,
        in_specs=[
            pl.BlockSpec((_BLK, _D), lambda i: (i, 0)),
            pl.BlockSpec((_BLK, _D), lambda i: (i, 0)),
            pl.BlockSpec((_BLK, 1), lambda i: (i, 0)),
            pl.BlockSpec((2, _D), lambda i: (0, 0)),
            pl.BlockSpec((1, _D), lambda i: (0, 0)),
            pl.BlockSpec((1, _D), lambda i: (0, 0)),
        ],
        out_specs=pl.BlockSpec((_BLK, _D), lambda i: (i, 0)),
        out_shape=jax.ShapeDtypeStruct((_N, _D), jnp.float32),
    )(inputs2d, pos2d, ttf, type_table, gamma2d, beta2d)


@jax.jit
def kernel(inputs_embeds, position_ids, token_type_ids, pos_table, type_table,
           ln_gamma, ln_beta):
    idx = position_ids.reshape(_N)
    pos2d = _sc_gather(pos_table, idx)
    inputs2d = inputs_embeds.reshape(_N, _D)
    ttf = token_type_ids.reshape(_N, 1).astype(jnp.float32)
    out2d = _tc_addln(inputs2d, pos2d, ttf, type_table,
                      ln_gamma.reshape(1, _D), ln_beta.reshape(1, _D))
    return out2d.reshape(_B, _S, _D)
